# baseline (speedup 1.0000x reference)
import functools

import jax
import jax.numpy as jnp
from jax import lax
from jax.experimental import pallas as pl
from jax.experimental.pallas import tpu as pltpu
from jax.experimental.pallas import tpu_sc as plsc

NC = 2
NS = 16
NW = NC * NS
CHUNK = 128
RING = 8
LAG = 4


@functools.lru_cache(maxsize=None)
def _sc_degree(n_pad: int, k_per_w: int):
    rows_per_sub = n_pad // NS
    mesh = plsc.VectorSubcoreMesh(core_axis_name="c", subcore_axis_name="s")

    @functools.partial(
        pl.kernel,
        out_type=jax.ShapeDtypeStruct((NC, n_pad), jnp.float32),
        mesh=mesh,
        scratch_types=[
            pltpu.VMEM((k_per_w, CHUNK), jnp.int32),
            pltpu.VMEM((CHUNK,), jnp.float32),
            pltpu.VMEM((rows_per_sub,), jnp.float32),
            pltpu.VMEM_SHARED((n_pad,), jnp.float32),
        ],
        compiler_params=pltpu.CompilerParams(use_tc_tiling_on_sc=False),
    )
    def deg_kernel(dst_hbm, out_hbm, idx_v, ones_v, zbuf_v, deg_sh):
        c = lax.axis_index("c")
        s = lax.axis_index("s")
        wid = c * NS + s
        one16 = jnp.ones((16,), jnp.float32)
        zero16 = jnp.zeros((16,), jnp.float32)
        for j in range(CHUNK // 16):
            ones_v[pl.ds(j * 16, 16)] = one16

        @pl.loop(0, rows_per_sub // 16)
        def _(i):
            zbuf_v[pl.ds(i * 16, 16)] = zero16

        pltpu.sync_copy(zbuf_v, deg_sh.at[pl.ds(s * rows_per_sub, rows_per_sub)])
        pltpu.sync_copy(dst_hbm.at[pl.ds(wid * k_per_w, k_per_w)], idx_v)
        plsc.subcore_barrier()

        @pl.loop(0, k_per_w)
        def _(j):
            pltpu.sync_copy(ones_v, deg_sh.at[idx_v.at[j]], add=True)

        plsc.subcore_barrier()
        pltpu.sync_copy(
            deg_sh.at[pl.ds(s * rows_per_sub, rows_per_sub)],
            out_hbm.at[c, pl.ds(s * rows_per_sub, rows_per_sub)],
        )

    return deg_kernel


@functools.lru_cache(maxsize=None)
def _sc_aggregate(n: int, n_pad: int, h: int, k_per_w: int):
    rows_per_sub = n_pad // NS
    mesh = plsc.VectorSubcoreMesh(core_axis_name="c", subcore_axis_name="s")
    assert k_per_w % RING == 0 and k_per_w >= RING + LAG

    @functools.partial(
        pl.kernel,
        out_type=jax.ShapeDtypeStruct((NC, n_pad, h), jnp.int16),
        mesh=mesh,
        scratch_types=[
            pltpu.VMEM((k_per_w, CHUNK), jnp.int32),
            pltpu.VMEM((k_per_w, CHUNK), jnp.int32),
            pltpu.VMEM((RING, CHUNK, h), jnp.int16),
            pltpu.VMEM((CHUNK, h), jnp.int16),
            pltpu.VMEM_SHARED((n_pad, h), jnp.int16),
            [pltpu.SemaphoreType.DMA] * RING,
            [pltpu.SemaphoreType.DMA] * RING,
        ],
        compiler_params=pltpu.CompilerParams(use_tc_tiling_on_sc=False),
    )
    def agg_kernel(u_hbm, src_hbm, dst_hbm, out_hbm,
                   src_v, dst_v, rows_v, zbuf_v, agg_sh, gsems, ssems):
        c = lax.axis_index("c")
        s = lax.axis_index("s")
        wid = c * NS + s
        zero32 = jnp.zeros((32,), jnp.int16)

        def gather_start(j, buf):
            pltpu.async_copy(u_hbm.at[src_v.at[j]], rows_v.at[buf], gsems[buf])

        def gather_wait(j, buf):
            pltpu.make_async_copy(
                u_hbm.at[src_v.at[j]], rows_v.at[buf], gsems[buf]
            ).wait()

        def scatter_start(j, buf):
            pltpu.async_copy(
                rows_v.at[buf], agg_sh.at[dst_v.at[j]], ssems[buf], add=True
            )

        def scatter_wait(j, buf):
            pltpu.make_async_copy(
                rows_v.at[buf], agg_sh.at[dst_v.at[j]], ssems[buf]
            ).wait()

        @pl.loop(0, CHUNK)
        def _(i):
            for j in range(h // 32):
                zbuf_v[i, pl.ds(j * 32, 32)] = zero32

        for r in range(rows_per_sub // CHUNK):
            pltpu.sync_copy(
                zbuf_v, agg_sh.at[pl.ds(s * rows_per_sub + r * CHUNK, CHUNK)]
            )
        pltpu.sync_copy(src_hbm.at[pl.ds(wid * k_per_w, k_per_w)], src_v)
        pltpu.sync_copy(dst_hbm.at[pl.ds(wid * k_per_w, k_per_w)], dst_v)
        plsc.subcore_barrier()

        for b in range(RING):
            gather_start(b, b)
        for b in range(LAG):
            gather_wait(b, b)
            scatter_start(b, b)

        @pl.loop(0, (k_per_w - RING) // RING)
        def _(t):
            j0 = LAG + t * RING
            for b in range(RING):
                j = j0 + b
                bb = (LAG + b) % RING
                bn = (bb + LAG) % RING
                scatter_wait(j - LAG, bn)
                gather_start(j + LAG, bn)
                gather_wait(j, bb)
                scatter_start(j, bb)

        for b in range(LAG):
            j = k_per_w - LAG + b
            bb = (LAG + b) % RING
            gather_wait(j, bb)
            scatter_start(j, bb)
        for b in range(RING):
            j = k_per_w - RING + b
            scatter_wait(j, b)

        plsc.subcore_barrier()
        pltpu.sync_copy(
            agg_sh.at[pl.ds(s * rows_per_sub, rows_per_sub)],
            out_hbm.at[c, pl.ds(s * rows_per_sub, rows_per_sub)],
        )

    return agg_kernel


def _quantize(u, maxdeg):
    m = jnp.max(jnp.abs(u))
    step = jnp.maximum(m * maxdeg * (1.0 / 32767.0), 1e-30)
    q = jnp.clip(jnp.round(u * (1.0 / step)), -32767.0, 32767.0)
    return q.astype(jnp.int16), step


def _tc_entry(deg_p, x, w0):
    n = x.shape[0]

    def body(deg_ref, x_ref, w_ref, uq_ref, dinv_ref, step_ref, mdeg_ref):
        deg_e = deg_ref[0, :n] + deg_ref[1, :n]
        maxdeg = jnp.maximum(jnp.max(deg_e), 1.0)
        mdeg_ref[...] = jnp.full((1, 1), maxdeg, jnp.float32)
        dinv = lax.rsqrt(deg_e + 1.0)
        dinv_ref[...] = dinv
        z = jnp.dot(x_ref[...], w_ref[...], preferred_element_type=jnp.float32)
        u = z * dinv[:, None]
        q, step = _quantize(u, maxdeg)
        uq_ref[...] = q
        step_ref[...] = jnp.full((1, 1), step, jnp.float32)

    return pl.pallas_call(
        body,
        out_shape=(
            jax.ShapeDtypeStruct((n, w0.shape[1]), jnp.int16),
            jax.ShapeDtypeStruct((n,), jnp.float32),
            jax.ShapeDtypeStruct((1, 1), jnp.float32),
            jax.ShapeDtypeStruct((1, 1), jnp.float32),
        ),
    )(deg_p, x, w0)


def _tc_layer(agg_p, uq, step, mdeg, dinv, b, w_next):
    n, h = uq.shape

    def body(agg_ref, uq_ref, step_ref, mdeg_ref, dinv_ref, b_ref, w_ref,
             uq_out_ref, step_out_ref):
        stp = step_ref[0, 0]
        dinv_col = dinv_ref[...][:, None]
        tot = (agg_ref[0, :n, :].astype(jnp.float32)
               + agg_ref[1, :n, :].astype(jnp.float32)
               + uq_ref[...].astype(jnp.float32)) * stp
        hid = jnp.maximum(tot * dinv_col + b_ref[...][None, :], 0.0)
        z = jnp.dot(hid, w_ref[...], preferred_element_type=jnp.float32)
        u = z * dinv_col
        q, step_out = _quantize(u, mdeg_ref[0, 0])
        uq_out_ref[...] = q
        step_out_ref[...] = jnp.full((1, 1), step_out, jnp.float32)

    return pl.pallas_call(
        body,
        out_shape=(
            jax.ShapeDtypeStruct((n, h), jnp.int16),
            jax.ShapeDtypeStruct((1, 1), jnp.float32),
        ),
    )(agg_p, uq, step, mdeg, dinv, b, w_next)


def _tc_final(agg_p, uq, step, dinv, b):
    n, h = uq.shape

    def body(agg_ref, uq_ref, step_ref, dinv_ref, b_ref, out_ref):
        stp = step_ref[0, 0]
        dinv_col = dinv_ref[...][:, None]
        tot = (agg_ref[0, :n, :].astype(jnp.float32)
               + agg_ref[1, :n, :].astype(jnp.float32)
               + uq_ref[...].astype(jnp.float32)) * stp
        out_ref[...] = jnp.maximum(tot * dinv_col + b_ref[...][None, :], 0.0)

    return pl.pallas_call(
        body,
        out_shape=jax.ShapeDtypeStruct((n, h), jnp.float32),
    )(agg_p, uq, step, dinv, b)


def kernel(x, edge_index, batch_index, W0, b0, W1, b1, W2, b2, W3, b3):
    n, _ = x.shape
    h = W0.shape[1]
    e = edge_index.shape[1]

    e_pad = -(-e // (NW * CHUNK * RING)) * (NW * CHUNK * RING)
    n_pad = -(-(n + 1) // (NS * CHUNK)) * (NS * CHUNK)
    k_per_w = e_pad // (NW * CHUNK)

    src = edge_index[0].astype(jnp.int32)
    dst = edge_index[1].astype(jnp.int32)
    src2d = jnp.concatenate(
        [src, jnp.zeros((e_pad - e,), jnp.int32)]
    ).reshape(e_pad // CHUNK, CHUNK)
    junk = n + jnp.arange(e_pad - e, dtype=jnp.int32) % (n_pad - n)
    dst2d = jnp.concatenate([dst, junk]).reshape(e_pad // CHUNK, CHUNK)

    deg_p = _sc_degree(n_pad, k_per_w)(dst2d)
    uq, dinv, step, mdeg = _tc_entry(deg_p, x, W0)
    agg = _sc_aggregate(n, n_pad, h, k_per_w)
    for b, w_next in ((b0, W1), (b1, W2), (b2, W3)):
        agg_p = agg(uq, src2d, dst2d)
        uq, step = _tc_layer(agg_p, uq, step, mdeg, dinv, b, w_next)
    agg_p = agg(uq, src2d, dst2d)
    return _tc_final(agg_p, uq, step, dinv, b3)


# speedup vs baseline: 1.0022x; 1.0022x over previous
import functools

import jax
import jax.numpy as jnp
from jax import lax
from jax.experimental import pallas as pl
from jax.experimental.pallas import tpu as pltpu
from jax.experimental.pallas import tpu_sc as plsc

NC = 2
NS = 16
NW = NC * NS
CHUNK = 128
RING = 8
LAG = 4


@functools.lru_cache(maxsize=None)
def _sc_degree(n_pad: int, k_per_w: int):
    rows_per_sub = n_pad // NS
    mesh = plsc.VectorSubcoreMesh(core_axis_name="c", subcore_axis_name="s")

    @functools.partial(
        pl.kernel,
        out_type=jax.ShapeDtypeStruct((NC, n_pad), jnp.float32),
        mesh=mesh,
        scratch_types=[
            pltpu.VMEM((k_per_w, CHUNK), jnp.int32),
            pltpu.VMEM((CHUNK,), jnp.float32),
            pltpu.VMEM((rows_per_sub,), jnp.float32),
            pltpu.VMEM_SHARED((n_pad,), jnp.float32),
        ],
        compiler_params=pltpu.CompilerParams(use_tc_tiling_on_sc=False),
    )
    def deg_kernel(dst_hbm, out_hbm, idx_v, ones_v, zbuf_v, deg_sh):
        c = lax.axis_index("c")
        s = lax.axis_index("s")
        wid = c * NS + s
        one16 = jnp.ones((16,), jnp.float32)
        zero16 = jnp.zeros((16,), jnp.float32)
        for j in range(CHUNK // 16):
            ones_v[pl.ds(j * 16, 16)] = one16

        @pl.loop(0, rows_per_sub // 16)
        def _(i):
            zbuf_v[pl.ds(i * 16, 16)] = zero16

        pltpu.sync_copy(zbuf_v, deg_sh.at[pl.ds(s * rows_per_sub, rows_per_sub)])
        pltpu.sync_copy(dst_hbm.at[pl.ds(wid * k_per_w, k_per_w)], idx_v)
        plsc.subcore_barrier()

        @pl.loop(0, k_per_w)
        def _(j):
            pltpu.sync_copy(ones_v, deg_sh.at[idx_v.at[j]], add=True)

        plsc.subcore_barrier()
        pltpu.sync_copy(
            deg_sh.at[pl.ds(s * rows_per_sub, rows_per_sub)],
            out_hbm.at[c, pl.ds(s * rows_per_sub, rows_per_sub)],
        )

    return deg_kernel


@functools.lru_cache(maxsize=None)
def _sc_aggregate(n: int, n_pad: int, h: int, k_per_w: int):
    rows_per_sub = n_pad // NS
    mesh = plsc.VectorSubcoreMesh(core_axis_name="c", subcore_axis_name="s")
    assert k_per_w % RING == 0 and k_per_w >= RING + LAG

    @functools.partial(
        pl.kernel,
        out_type=jax.ShapeDtypeStruct((NC, n_pad, h), jnp.int16),
        mesh=mesh,
        scratch_types=[
            pltpu.VMEM((k_per_w, CHUNK), jnp.int32),
            pltpu.VMEM((k_per_w, CHUNK), jnp.int32),
            pltpu.VMEM((RING, CHUNK, h), jnp.int16),
            pltpu.VMEM((CHUNK, h), jnp.int16),
            pltpu.VMEM_SHARED((n_pad, h), jnp.int16),
            [pltpu.SemaphoreType.DMA] * RING,
            [pltpu.SemaphoreType.DMA] * RING,
        ],
        compiler_params=pltpu.CompilerParams(use_tc_tiling_on_sc=False),
    )
    def agg_kernel(u_hbm, src_hbm, dst_hbm, out_hbm,
                   src_v, dst_v, rows_v, zbuf_v, agg_sh, gsems, ssems):
        c = lax.axis_index("c")
        s = lax.axis_index("s")
        wid = c * NS + s
        zero32 = jnp.zeros((32,), jnp.int16)

        def gather_start(j, buf):
            pltpu.async_copy(u_hbm.at[src_v.at[j]], rows_v.at[buf], gsems[buf])

        def gather_wait(j, buf):
            pltpu.make_async_copy(
                u_hbm.at[src_v.at[j]], rows_v.at[buf], gsems[buf]
            ).wait()

        def scatter_start(j, buf):
            pltpu.async_copy(
                rows_v.at[buf], agg_sh.at[dst_v.at[j]], ssems[buf], add=True
            )

        def scatter_wait(j, buf):
            pltpu.make_async_copy(
                rows_v.at[buf], agg_sh.at[dst_v.at[j]], ssems[buf]
            ).wait()

        @pl.loop(0, CHUNK)
        def _(i):
            for j in range(h // 32):
                zbuf_v[i, pl.ds(j * 32, 32)] = zero32

        for r in range(rows_per_sub // CHUNK):
            pltpu.sync_copy(
                zbuf_v, agg_sh.at[pl.ds(s * rows_per_sub + r * CHUNK, CHUNK)]
            )
        pltpu.sync_copy(src_hbm.at[pl.ds(wid * k_per_w, k_per_w)], src_v)
        pltpu.sync_copy(dst_hbm.at[pl.ds(wid * k_per_w, k_per_w)], dst_v)
        plsc.subcore_barrier()

        for b in range(RING):
            gather_start(b, b)
        for b in range(LAG):
            gather_wait(b, b)
            scatter_start(b, b)

        @pl.loop(0, (k_per_w - RING) // RING)
        def _(t):
            j0 = LAG + t * RING
            for b in range(RING):
                j = j0 + b
                bb = (LAG + b) % RING
                bn = (bb + LAG) % RING
                scatter_wait(j - LAG, bn)
                gather_start(j + LAG, bn)
                gather_wait(j, bb)
                scatter_start(j, bb)

        for b in range(LAG):
            j = k_per_w - LAG + b
            bb = (LAG + b) % RING
            gather_wait(j, bb)
            scatter_start(j, bb)
        for b in range(RING):
            j = k_per_w - RING + b
            scatter_wait(j, b)

        plsc.subcore_barrier()
        pltpu.sync_copy(
            agg_sh.at[pl.ds(s * rows_per_sub, rows_per_sub)],
            out_hbm.at[c, pl.ds(s * rows_per_sub, rows_per_sub)],
        )

    return agg_kernel


def _quantize(u, maxdeg):
    m = jnp.max(jnp.abs(u))
    step = jnp.maximum(m * maxdeg * (1.0 / 32000.0), 1e-30)
    q = jnp.clip(jnp.round(u * (1.0 / step)), -32000.0, 32000.0)
    return q.astype(jnp.int16), step


def _tc_entry(deg_p, x, w0):
    n = x.shape[0]

    def body(deg_ref, x_ref, w_ref, uq_ref, dinv_ref, step_ref, mdeg_ref):
        deg_e = deg_ref[0, :n] + deg_ref[1, :n]
        maxdeg = jnp.maximum(jnp.max(deg_e), 1.0)
        mdeg_ref[...] = jnp.full((1, 1), maxdeg, jnp.float32)
        dinv = lax.rsqrt(deg_e + 1.0)
        dinv_ref[...] = dinv
        z = jnp.dot(x_ref[...], w_ref[...], preferred_element_type=jnp.float32)
        u = z * dinv[:, None]
        q, step = _quantize(u, maxdeg)
        uq_ref[...] = q
        step_ref[...] = jnp.full((1, 1), step, jnp.float32)

    return pl.pallas_call(
        body,
        out_shape=(
            jax.ShapeDtypeStruct((n, w0.shape[1]), jnp.int16),
            jax.ShapeDtypeStruct((n,), jnp.float32),
            jax.ShapeDtypeStruct((1, 1), jnp.float32),
            jax.ShapeDtypeStruct((1, 1), jnp.float32),
        ),
    )(deg_p, x, w0)


def _tc_layer(agg_p, uq, step, mdeg, dinv, b, w_next):
    n, h = uq.shape

    def body(agg_ref, uq_ref, step_ref, mdeg_ref, dinv_ref, b_ref, w_ref,
             uq_out_ref, step_out_ref):
        stp = step_ref[0, 0]
        dinv_col = dinv_ref[...][:, None]
        tot = (agg_ref[0, :n, :].astype(jnp.float32)
               + agg_ref[1, :n, :].astype(jnp.float32)
               + uq_ref[...].astype(jnp.float32)) * stp
        hid = jnp.maximum(tot * dinv_col + b_ref[...][None, :], 0.0)
        z = jnp.dot(hid, w_ref[...], preferred_element_type=jnp.float32)
        u = z * dinv_col
        q, step_out = _quantize(u, mdeg_ref[0, 0])
        uq_out_ref[...] = q
        step_out_ref[...] = jnp.full((1, 1), step_out, jnp.float32)

    return pl.pallas_call(
        body,
        out_shape=(
            jax.ShapeDtypeStruct((n, h), jnp.int16),
            jax.ShapeDtypeStruct((1, 1), jnp.float32),
        ),
    )(agg_p, uq, step, mdeg, dinv, b, w_next)


def _tc_final(agg_p, uq, step, dinv, b):
    n, h = uq.shape

    def body(agg_ref, uq_ref, step_ref, dinv_ref, b_ref, out_ref):
        stp = step_ref[0, 0]
        dinv_col = dinv_ref[...][:, None]
        tot = (agg_ref[0, :n, :].astype(jnp.float32)
               + agg_ref[1, :n, :].astype(jnp.float32)
               + uq_ref[...].astype(jnp.float32)) * stp
        out_ref[...] = jnp.maximum(tot * dinv_col + b_ref[...][None, :], 0.0)

    return pl.pallas_call(
        body,
        out_shape=jax.ShapeDtypeStruct((n, h), jnp.float32),
    )(agg_p, uq, step, dinv, b)


def kernel(x, edge_index, batch_index, W0, b0, W1, b1, W2, b2, W3, b3):
    n, _ = x.shape
    h = W0.shape[1]
    e = edge_index.shape[1]

    e_pad = -(-e // (NW * CHUNK * RING)) * (NW * CHUNK * RING)
    n_pad = -(-(n + 1) // (NS * CHUNK)) * (NS * CHUNK)
    k_per_w = e_pad // (NW * CHUNK)

    src = edge_index[0].astype(jnp.int32)
    dst = edge_index[1].astype(jnp.int32)
    src2d = jnp.concatenate(
        [src, jnp.zeros((e_pad - e,), jnp.int32)]
    ).reshape(e_pad // CHUNK, CHUNK)
    junk = n + jnp.arange(e_pad - e, dtype=jnp.int32) % (n_pad - n)
    dst2d = jnp.concatenate([dst, junk]).reshape(e_pad // CHUNK, CHUNK)

    deg_p = _sc_degree(n_pad, k_per_w)(dst2d)
    uq, dinv, step, mdeg = _tc_entry(deg_p, x, W0)
    agg = _sc_aggregate(n, n_pad, h, k_per_w)
    for b, w_next in ((b0, W1), (b1, W2), (b2, W3)):
        agg_p = agg(uq, src2d, dst2d)
        uq, step = _tc_layer(agg_p, uq, step, mdeg, dinv, b, w_next)
    agg_p = agg(uq, src2d, dst2d)
    return _tc_final(agg_p, uq, step, dinv, b3)
